# Initial kernel scaffold; baseline (speedup 1.0000x reference)
import jax
import jax.numpy as jnp
from jax.experimental import pallas as pl


def kernel(x, edge_index, W1, b1, W2, b2, W3, b3):
    raise NotImplementedError("write your pallas kernel here")


# speedup vs baseline: 10.4323x; 10.4323x over previous
import functools

import jax
import jax.numpy as jnp
from jax import lax
from jax.experimental import pallas as pl
from jax.experimental.pallas import tpu as pltpu
from jax.experimental.pallas import tpu_sc as plsc

N_NODES = 10000
D = 128
DH = D // 2
N_EDGES = 320000

NC = 2
NS = 16
CHUNK = 128
NPAD = 10240
EPAD = 327680
EPT32 = EPAD // (NC * NS)
EPT16 = EPAD // NS
NCHUNK = EPT16 // CHUNK
ROWS_PER_TILE = NPAD // NS

_MESH = plsc.VectorSubcoreMesh(core_axis_name="c", subcore_axis_name="s")
_SC_PARAMS = pltpu.CompilerParams(needs_layout_passes=False,
                                  use_tc_tiling_on_sc=False)


@functools.partial(
    pl.kernel,
    out_type=jax.ShapeDtypeStruct((NC * NS, NPAD), jnp.float32),
    mesh=_MESH,
    scratch_types=[
        pltpu.VMEM((EPT32,), jnp.int32),
        pltpu.VMEM((NPAD,), jnp.float32),
    ],
    compiler_params=_SC_PARAMS,
)
def _deg_kernel(dst_hbm, deg_out, idx_v, deg_v):
    c = lax.axis_index("c")
    s = lax.axis_index("s")
    wid = c * NS + s
    pltpu.sync_copy(dst_hbm.at[wid], idx_v)
    zeros16 = jnp.zeros((16,), jnp.float32)
    ones16 = jnp.ones((16,), jnp.float32)

    def _zero(i, carry):
        deg_v[pl.ds(i * 16, 16)] = zeros16
        return carry

    lax.fori_loop(0, NPAD // 16, _zero, 0)

    def _count(i, carry):
        idx = idx_v[pl.ds(i * 16, 16)]
        plsc.addupdate_scatter(deg_v, [idx], ones16)
        return carry

    lax.fori_loop(0, EPT32 // 16, _count, 0)
    pltpu.sync_copy(deg_v, deg_out.at[wid])


@functools.partial(
    pl.kernel,
    out_type=jax.ShapeDtypeStruct((NC, NPAD, DH), jnp.float32),
    mesh=_MESH,
    scratch_types=[
        pltpu.VMEM((NCHUNK, CHUNK), jnp.int32),
        pltpu.VMEM((NCHUNK, CHUNK), jnp.int32),
        pltpu.VMEM((2, CHUNK, DH), jnp.float32),
        pltpu.VMEM_SHARED((NPAD, DH), jnp.float32),
        pltpu.SemaphoreType.DMA,
        pltpu.SemaphoreType.DMA,
    ],
    compiler_params=_SC_PARAMS,
)
def _segsum_kernel(y_hbm, src_hbm, dst_hbm, acc_out,
                   src_v, dst_v, rows_v, acc_sh, gsem, gsem2):
    c = lax.axis_index("c")
    s = lax.axis_index("s")
    pltpu.sync_copy(src_hbm.at[s], src_v)
    pltpu.sync_copy(dst_hbm.at[s], dst_v)

    zeros16 = jnp.zeros((16,), jnp.float32)

    def _zero(i, carry):
        r = i // (DH // 16)
        k = i % (DH // 16)
        rows_v[0, r, pl.ds(k * 16, 16)] = zeros16
        return carry

    lax.fori_loop(0, CHUNK * DH // 16, _zero, 0)

    base = s * ROWS_PER_TILE
    for z in range(ROWS_PER_TILE // CHUNK):
        pltpu.sync_copy(rows_v.at[0], acc_sh.at[pl.ds(base + z * CHUNK, CHUNK)])
    plsc.subcore_barrier()

    yc = y_hbm.at[c]
    sems = (gsem, gsem2)
    pltpu.async_copy(yc.at[src_v.at[0]], rows_v.at[0], sems[0])

    def _edges(g, carry):
        for b in range(2):
            j = 2 * g + b
            jn = jnp.minimum(j + 1, NCHUNK - 1)
            pltpu.async_copy(yc.at[src_v.at[jn]], rows_v.at[1 - b],
                             sems[1 - b])
            pltpu.make_async_copy(yc.at[src_v.at[j]], rows_v.at[b],
                                  sems[b]).wait()
            pltpu.sync_copy(rows_v.at[b], acc_sh.at[dst_v.at[j]], add=True)
        return carry

    lax.fori_loop(0, NCHUNK // 2, _edges, 0)
    pltpu.make_async_copy(yc.at[src_v.at[0]], rows_v.at[0], sems[0]).wait()
    plsc.subcore_barrier()

    pltpu.sync_copy(acc_sh.at[pl.ds(base, ROWS_PER_TILE)],
                    acc_out.at[c, pl.ds(base, ROWS_PER_TILE)])


BLK = 512


def _dinv_body(parts_ref, dinv_ref):
    deg = jnp.sum(parts_ref[...], axis=0) + 1.0
    dinv_ref[...] = lax.rsqrt(deg)


def _dinv(parts):
    return pl.pallas_call(
        _dinv_body,
        out_shape=jax.ShapeDtypeStruct((NPAD // D, D), jnp.float32),
    )(parts)


def _y_body(x_ref, w_ref, dinv_ref, y_ref):
    xw = jnp.dot(x_ref[...], w_ref[...], preferred_element_type=jnp.float32)
    y = dinv_ref[...] * xw
    y_ref[0] = y[:, :DH]
    y_ref[1] = y[:, DH:]


def _y_pass(x, w, dinv_col):
    grid = (NPAD // BLK,)
    return pl.pallas_call(
        _y_body,
        grid=grid,
        in_specs=[
            pl.BlockSpec((BLK, D), lambda i: (i, 0)),
            pl.BlockSpec((D, D), lambda i: (0, 0)),
            pl.BlockSpec((BLK, 1), lambda i: (i, 0)),
        ],
        out_specs=pl.BlockSpec((NC, BLK, DH), lambda i: (0, i, 0)),
        out_shape=jax.ShapeDtypeStruct((NC, NPAD, DH), jnp.float32),
    )(x, w, dinv_col)


def _combine_body(relu, acc_ref, y_ref, dinv_ref, b_ref, o_ref):
    dinv = dinv_ref[...]
    b = b_ref[...]
    lo = dinv * (acc_ref[0] + y_ref[0]) + b[:, :DH]
    hi = dinv * (acc_ref[1] + y_ref[1]) + b[:, DH:]
    t = jnp.concatenate([lo, hi], axis=1)
    if relu:
        t = jnp.maximum(t, 0.0)
    o_ref[...] = t


def _combine(acc, y, dinv_col, b, relu):
    grid = (NPAD // BLK,)
    return pl.pallas_call(
        functools.partial(_combine_body, relu),
        grid=grid,
        in_specs=[
            pl.BlockSpec((NC, BLK, DH), lambda i: (0, i, 0)),
            pl.BlockSpec((NC, BLK, DH), lambda i: (0, i, 0)),
            pl.BlockSpec((BLK, 1), lambda i: (i, 0)),
            pl.BlockSpec((1, D), lambda i: (0, 0)),
        ],
        out_specs=pl.BlockSpec((BLK, D), lambda i: (i, 0)),
        out_shape=jax.ShapeDtypeStruct((NPAD, D), jnp.float32),
    )(acc, y, dinv_col, b)


def kernel(x, edge_index, W1, b1, W2, b2, W3, b3):
    src = edge_index[0].astype(jnp.int32)
    dst = edge_index[1].astype(jnp.int32)
    epad = EPAD - N_EDGES
    src_p = jnp.concatenate([src, jnp.zeros((epad,), jnp.int32)])
    dst_p = jnp.concatenate([dst, jnp.full((epad,), NPAD - 1, jnp.int32)])
    src16 = src_p.reshape(NS, NCHUNK, CHUNK)
    dst16 = dst_p.reshape(NS, NCHUNK, CHUNK)
    dst32 = dst_p.reshape(NC * NS, EPT32)

    x_pad = jnp.pad(x, ((0, NPAD - N_NODES), (0, 0)))

    deg_parts = _deg_kernel(dst32)
    dinv2d = _dinv(deg_parts.reshape(NC * NS, NPAD // D, D))
    dinv_col = dinv2d.reshape(NPAD, 1)

    h = x_pad
    for W, b, relu in ((W1, b1, True), (W2, b2, True), (W3, b3, False)):
        y = _y_pass(h, W, dinv_col)
        acc = _segsum_kernel(y, src16, dst16)
        h = _combine(acc, y, dinv_col, b.reshape(1, D), relu)
    return h[:N_NODES]


# speedup vs baseline: 10.5452x; 1.0108x over previous
import functools

import jax
import jax.numpy as jnp
from jax import lax
from jax.experimental import pallas as pl
from jax.experimental.pallas import tpu as pltpu
from jax.experimental.pallas import tpu_sc as plsc

N_NODES = 10000
D = 128
DH = D // 2
N_EDGES = 320000

NC = 2
NS = 16
CHUNK = 128
NPAD = 10240
EPAD = 327680
EPT32 = EPAD // (NC * NS)
EPT16 = EPAD // NS
NCHUNK = EPT16 // CHUNK
ROWS_PER_TILE = NPAD // NS
DEG_CHUNK = 2048

_MESH = plsc.VectorSubcoreMesh(core_axis_name="c", subcore_axis_name="s")
_SC_PARAMS = pltpu.CompilerParams(needs_layout_passes=False,
                                  use_tc_tiling_on_sc=False)


@functools.partial(
    pl.kernel,
    out_type=jax.ShapeDtypeStruct((NC * NS, NPAD), jnp.float32),
    mesh=_MESH,
    scratch_types=[
        pltpu.VMEM((DEG_CHUNK,), jnp.int32),
        pltpu.VMEM((NPAD,), jnp.float32),
    ],
    compiler_params=_SC_PARAMS,
)
def _deg_kernel(dst_hbm, deg_out, idx_v, deg_v):
    c = lax.axis_index("c")
    s = lax.axis_index("s")
    wid = c * NS + s
    zeros16 = jnp.zeros((16,), jnp.float32)
    ones16 = jnp.ones((16,), jnp.float32)

    def _zero(i, carry):
        deg_v[pl.ds(i * 16, 16)] = zeros16
        return carry

    lax.fori_loop(0, NPAD // 16, _zero, 0)

    def _count(i, carry):
        idx = idx_v[pl.ds(i * 16, 16)]
        plsc.addupdate_scatter(deg_v, [idx], ones16)
        return carry

    for ci in range(EPT32 // DEG_CHUNK):
        pltpu.sync_copy(dst_hbm.at[wid, pl.ds(ci * DEG_CHUNK, DEG_CHUNK)],
                        idx_v)
        lax.fori_loop(0, DEG_CHUNK // 16, _count, 0)
    pltpu.sync_copy(deg_v, deg_out.at[wid])


@functools.partial(
    pl.kernel,
    out_type=jax.ShapeDtypeStruct((NC, NPAD, DH), jnp.float32),
    mesh=_MESH,
    scratch_types=[
        pltpu.VMEM((NCHUNK, CHUNK), jnp.int32),
        pltpu.VMEM((NCHUNK, CHUNK), jnp.int32),
        pltpu.VMEM((4, CHUNK, DH), jnp.float32),
        pltpu.VMEM_SHARED((NPAD, DH), jnp.float32),
        pltpu.SemaphoreType.DMA,
        pltpu.SemaphoreType.DMA,
        pltpu.SemaphoreType.DMA,
        pltpu.SemaphoreType.DMA,
        pltpu.SemaphoreType.DMA,
        pltpu.SemaphoreType.DMA,
        pltpu.SemaphoreType.DMA,
        pltpu.SemaphoreType.DMA,
    ],
    compiler_params=_SC_PARAMS,
)
def _segsum_kernel(y_hbm, src_hbm, dst_hbm, acc_out,
                   src_v, dst_v, rows_v, acc_sh,
                   g0, g1, g2, g3, s0, s1, s2, s3):
    c = lax.axis_index("c")
    s = lax.axis_index("s")
    gsem = (g0, g1, g2, g3)
    ssem = (s0, s1, s2, s3)
    pltpu.sync_copy(src_hbm.at[s], src_v)
    pltpu.sync_copy(dst_hbm.at[s], dst_v)

    zeros16 = jnp.zeros((16,), jnp.float32)

    def _zero(i, carry):
        r = i // (DH // 16)
        k = i % (DH // 16)
        rows_v[0, r, pl.ds(k * 16, 16)] = zeros16
        return carry

    lax.fori_loop(0, CHUNK * DH // 16, _zero, 0)

    base = s * ROWS_PER_TILE
    for z in range(ROWS_PER_TILE // CHUNK):
        pltpu.sync_copy(rows_v.at[0], acc_sh.at[pl.ds(base + z * CHUNK, CHUNK)])
    plsc.subcore_barrier()

    yc = y_hbm.at[c]

    def _gather(j, b):
        pltpu.async_copy(yc.at[src_v.at[j]], rows_v.at[b], gsem[b])

    def _wait_gather(b):
        pltpu.make_async_copy(yc.at[src_v.at[0]], rows_v.at[b],
                              gsem[b]).wait()

    def _scatter(j, b):
        pltpu.async_copy(rows_v.at[b], acc_sh.at[dst_v.at[j]], ssem[b],
                         add=True)

    def _wait_scatter(b):
        pltpu.make_async_copy(rows_v.at[b], acc_sh.at[dst_v.at[0]],
                              ssem[b]).wait()

    _gather(0, 0)
    _gather(1, 1)
    for k in (0, 1):
        _wait_gather(k)
        _scatter(k, k)
        _gather(k + 2, k + 2)

    def _steady(g, carry):
        for bb in range(4):
            k = 2 + g * 4 + bb
            b = (2 + bb) % 4
            _wait_gather(b)
            _scatter(k, b)
            _wait_scatter((b + 2) % 4)
            _gather(k + 2, (b + 2) % 4)
        return carry

    lax.fori_loop(0, (NCHUNK - 4) // 4, _steady, 0)

    for k in (NCHUNK - 2, NCHUNK - 1):
        b = k % 4
        _wait_gather(b)
        _scatter(k, b)
        _wait_scatter((b + 2) % 4)
    _wait_scatter((NCHUNK - 2) % 4)
    _wait_scatter((NCHUNK - 1) % 4)
    plsc.subcore_barrier()

    pltpu.sync_copy(acc_sh.at[pl.ds(base, ROWS_PER_TILE)],
                    acc_out.at[c, pl.ds(base, ROWS_PER_TILE)])


BLK = 512


def _dinv_body(parts_ref, dinv_ref):
    deg = jnp.sum(parts_ref[...], axis=0) + 1.0
    dinv_ref[...] = lax.rsqrt(deg)


def _dinv(parts):
    return pl.pallas_call(
        _dinv_body,
        out_shape=jax.ShapeDtypeStruct((NPAD // D, D), jnp.float32),
    )(parts)


def _xw_body(x_ref, w_ref, o_ref):
    xw = jnp.dot(x_ref[...], w_ref[...], preferred_element_type=jnp.float32)
    o_ref[0] = xw[:, :DH]
    o_ref[1] = xw[:, DH:]


def _xw_pass(x, w):
    return pl.pallas_call(
        _xw_body,
        grid=(NPAD // BLK,),
        in_specs=[
            pl.BlockSpec((BLK, D), lambda i: (i, 0)),
            pl.BlockSpec((D, D), lambda i: (0, 0)),
        ],
        out_specs=pl.BlockSpec((NC, BLK, DH), lambda i: (0, i, 0)),
        out_shape=jax.ShapeDtypeStruct((NC, NPAD, DH), jnp.float32),
    )(x, w)


def _scale_body(xw_ref, dinv_ref, y_ref):
    dinv = dinv_ref[...]
    y_ref[0] = dinv * xw_ref[0]
    y_ref[1] = dinv * xw_ref[1]


def _scale(xw, dinv_col):
    return pl.pallas_call(
        _scale_body,
        grid=(NPAD // BLK,),
        in_specs=[
            pl.BlockSpec((NC, BLK, DH), lambda i: (0, i, 0)),
            pl.BlockSpec((BLK, 1), lambda i: (i, 0)),
        ],
        out_specs=pl.BlockSpec((NC, BLK, DH), lambda i: (0, i, 0)),
        out_shape=jax.ShapeDtypeStruct((NC, NPAD, DH), jnp.float32),
    )(xw, dinv_col)


def _cm_body(acc_ref, y_ref, dinv_ref, b_ref, w_ref, o_ref):
    dinv = dinv_ref[...]
    b = b_ref[...]
    lo = dinv * (acc_ref[0] + y_ref[0]) + b[:, :DH]
    hi = dinv * (acc_ref[1] + y_ref[1]) + b[:, DH:]
    h = jnp.maximum(jnp.concatenate([lo, hi], axis=1), 0.0)
    y = dinv * jnp.dot(h, w_ref[...], preferred_element_type=jnp.float32)
    o_ref[0] = y[:, :DH]
    o_ref[1] = y[:, DH:]


def _combine_matmul_scale(acc, y, dinv_col, b, w):
    return pl.pallas_call(
        _cm_body,
        grid=(NPAD // BLK,),
        in_specs=[
            pl.BlockSpec((NC, BLK, DH), lambda i: (0, i, 0)),
            pl.BlockSpec((NC, BLK, DH), lambda i: (0, i, 0)),
            pl.BlockSpec((BLK, 1), lambda i: (i, 0)),
            pl.BlockSpec((1, D), lambda i: (0, 0)),
            pl.BlockSpec((D, D), lambda i: (0, 0)),
        ],
        out_specs=pl.BlockSpec((NC, BLK, DH), lambda i: (0, i, 0)),
        out_shape=jax.ShapeDtypeStruct((NC, NPAD, DH), jnp.float32),
    )(acc, y, dinv_col, b, w)


def _final_body(acc_ref, y_ref, dinv_ref, b_ref, o_ref):
    dinv = dinv_ref[...]
    b = b_ref[...]
    lo = dinv * (acc_ref[0] + y_ref[0]) + b[:, :DH]
    hi = dinv * (acc_ref[1] + y_ref[1]) + b[:, DH:]
    o_ref[...] = jnp.concatenate([lo, hi], axis=1)


def _final(acc, y, dinv_col, b):
    return pl.pallas_call(
        _final_body,
        grid=(NPAD // BLK,),
        in_specs=[
            pl.BlockSpec((NC, BLK, DH), lambda i: (0, i, 0)),
            pl.BlockSpec((NC, BLK, DH), lambda i: (0, i, 0)),
            pl.BlockSpec((BLK, 1), lambda i: (i, 0)),
            pl.BlockSpec((1, D), lambda i: (0, 0)),
        ],
        out_specs=pl.BlockSpec((BLK, D), lambda i: (i, 0)),
        out_shape=jax.ShapeDtypeStruct((NPAD, D), jnp.float32),
    )(acc, y, dinv_col, b)


def kernel(x, edge_index, W1, b1, W2, b2, W3, b3):
    src = edge_index[0].astype(jnp.int32)
    dst = edge_index[1].astype(jnp.int32)
    epad = EPAD - N_EDGES
    src_p = jnp.concatenate([src, jnp.zeros((epad,), jnp.int32)])
    dst_p = jnp.concatenate([dst, jnp.full((epad,), NPAD - 1, jnp.int32)])
    src16 = src_p.reshape(NS, NCHUNK, CHUNK)
    dst16 = dst_p.reshape(NS, NCHUNK, CHUNK)
    dst32 = dst_p.reshape(NC * NS, EPT32)

    x_pad = jnp.pad(x, ((0, NPAD - N_NODES), (0, 0)))

    xw1 = _xw_pass(x_pad, W1)
    deg_parts = _deg_kernel(dst32)
    dinv2d = _dinv(deg_parts.reshape(NC * NS, NPAD // D, D))
    dinv_col = dinv2d.reshape(NPAD, 1)

    y = _scale(xw1, dinv_col)
    acc = _segsum_kernel(y, src16, dst16)
    y = _combine_matmul_scale(acc, y, dinv_col, b1.reshape(1, D), W2)
    acc = _segsum_kernel(y, src16, dst16)
    y = _combine_matmul_scale(acc, y, dinv_col, b2.reshape(1, D), W3)
    acc = _segsum_kernel(y, src16, dst16)
    h = _final(acc, y, dinv_col, b3.reshape(1, D))
    return h[:N_NODES]


# speedup vs baseline: 18.6164x; 1.7654x over previous
import functools

import jax
import jax.numpy as jnp
from jax import lax
from jax.experimental import pallas as pl
from jax.experimental.pallas import tpu as pltpu
from jax.experimental.pallas import tpu_sc as plsc

N_NODES = 10000
D = 128
DH = D // 2
N_EDGES = 320000

NC = 2
NS = 16
CHUNK = 128
NPAD = 10240
EPAD = 327680
EPT32 = EPAD // (NC * NS)
EPT16 = EPAD // NS
NCHUNK = EPT16 // CHUNK
ROWS_PER_TILE = NPAD // NS
DEG_CHUNK = 2048

_MESH = plsc.VectorSubcoreMesh(core_axis_name="c", subcore_axis_name="s")
_SC_PARAMS = pltpu.CompilerParams(needs_layout_passes=False,
                                  use_tc_tiling_on_sc=False)


@functools.partial(
    pl.kernel,
    out_type=jax.ShapeDtypeStruct((NC * NS, NPAD), jnp.float32),
    mesh=_MESH,
    scratch_types=[
        pltpu.VMEM((DEG_CHUNK,), jnp.int32),
        pltpu.VMEM((NPAD,), jnp.float32),
    ],
    compiler_params=_SC_PARAMS,
)
def _deg_kernel(dst_hbm, deg_out, idx_v, deg_v):
    c = lax.axis_index("c")
    s = lax.axis_index("s")
    wid = c * NS + s
    zeros16 = jnp.zeros((16,), jnp.float32)
    ones16 = jnp.ones((16,), jnp.float32)

    def _zero(i, carry):
        deg_v[pl.ds(i * 16, 16)] = zeros16
        return carry

    lax.fori_loop(0, NPAD // 16, _zero, 0)

    def _count(i, carry):
        idx = idx_v[pl.ds(i * 16, 16)]
        plsc.addupdate_scatter(deg_v, [idx], ones16)
        return carry

    for ci in range(EPT32 // DEG_CHUNK):
        pltpu.sync_copy(dst_hbm.at[wid, pl.ds(ci * DEG_CHUNK, DEG_CHUNK)],
                        idx_v)
        lax.fori_loop(0, DEG_CHUNK // 16, _count, 0)
    pltpu.sync_copy(deg_v, deg_out.at[wid])


PHCH = 20
NPHASE = NCHUNK // PHCH


@functools.partial(
    pl.kernel,
    out_type=jax.ShapeDtypeStruct((NC, NPAD, DH), jnp.float32),
    mesh=_MESH,
    scratch_types=[
        pltpu.VMEM((2, PHCH, CHUNK), jnp.int32),
        pltpu.VMEM((2, PHCH, CHUNK), jnp.int32),
        pltpu.VMEM((2, CHUNK, DH), jnp.float32),
        pltpu.VMEM_SHARED((NPAD, DH), jnp.float32),
        pltpu.VMEM_SHARED((NPAD, DH), jnp.float32),
        pltpu.SemaphoreType.DMA,
        pltpu.SemaphoreType.DMA,
        pltpu.SemaphoreType.DMA,
        pltpu.SemaphoreType.DMA,
    ],
    compiler_params=_SC_PARAMS,
)
def _segsum_kernel(y_hbm, src_hbm, dst_hbm, acc_out,
                   src_v, dst_v, rows_v, y_sh, acc_sh,
                   g0, g1, i0, i1):
    c = lax.axis_index("c")
    s = lax.axis_index("s")
    gsem = (g0, g1)
    isem = (i0, i1)

    def _idx_fetch(p, pb):
        pltpu.async_copy(src_hbm.at[s, pl.ds(p * PHCH, PHCH)],
                         src_v.at[pb], isem[pb])
        pltpu.async_copy(dst_hbm.at[s, pl.ds(p * PHCH, PHCH)],
                         dst_v.at[pb], isem[pb])

    def _idx_wait(pb):
        pltpu.make_async_copy(src_hbm.at[s, pl.ds(0, PHCH)],
                              src_v.at[pb], isem[pb]).wait()
        pltpu.make_async_copy(dst_hbm.at[s, pl.ds(0, PHCH)],
                              dst_v.at[pb], isem[pb]).wait()

    _idx_fetch(0, 0)
    _idx_fetch(1, 1)

    zeros16 = jnp.zeros((16,), jnp.float32)

    def _zero(i, carry):
        r = i // (DH // 16)
        k = i % (DH // 16)
        rows_v[0, r, pl.ds(k * 16, 16)] = zeros16
        return carry

    lax.fori_loop(0, CHUNK * DH // 16, _zero, 0)

    base = s * ROWS_PER_TILE
    for z in range(ROWS_PER_TILE // CHUNK):
        pltpu.sync_copy(rows_v.at[0], acc_sh.at[pl.ds(base + z * CHUNK, CHUNK)])
    pltpu.sync_copy(y_hbm.at[c, pl.ds(base, ROWS_PER_TILE)],
                    y_sh.at[pl.ds(base, ROWS_PER_TILE)])
    plsc.subcore_barrier()

    def _gather(idx_row, b):
        pltpu.async_copy(y_sh.at[idx_row], rows_v.at[b], gsem[b])

    def _wait_gather(b):
        pltpu.make_async_copy(y_sh.at[src_v.at[0, 0]], rows_v.at[b],
                              gsem[b]).wait()

    def _chunk(b, idx_jj, next_row):
        _gather(next_row, 1 - b)
        _wait_gather(b)
        pltpu.sync_copy(rows_v.at[b], acc_sh.at[idx_jj[1]], add=True)

    def _pair(t, pb):
        _chunk(0, (None, dst_v.at[pb, 2 * t]), src_v.at[pb, 2 * t + 1])
        _chunk(1, (None, dst_v.at[pb, 2 * t + 1]), src_v.at[pb, 2 * t + 2])

    def _phase_tail(pb, next_first_row):
        _chunk(0, (None, dst_v.at[pb, PHCH - 2]), src_v.at[pb, PHCH - 1])
        _chunk(1, (None, dst_v.at[pb, PHCH - 1]), next_first_row)

    def _phase(p, pb, last):
        if not last:
            _idx_wait(1 - pb)
        lax.fori_loop(0, PHCH // 2 - 1,
                      lambda t, carry: (_pair(t, pb), carry)[1], 0)
        if last:
            _phase_tail(pb, src_v.at[pb, PHCH - 1])
        else:
            _phase_tail(pb, src_v.at[1 - pb, 0])

    _idx_wait(0)
    _gather(src_v.at[0, 0], 0)

    def _superphase(q, carry):
        p0 = 2 * q
        _phase(p0, 0, False)
        _idx_fetch(p0 + 2, 0)
        _phase(p0 + 1, 1, False)
        _idx_fetch(p0 + 3, 1)
        return carry

    lax.fori_loop(0, (NPHASE - 2) // 2, _superphase, 0)
    _phase(NPHASE - 2, 0, False)
    _phase(NPHASE - 1, 1, True)
    _wait_gather(0)
    plsc.subcore_barrier()

    pltpu.sync_copy(acc_sh.at[pl.ds(base, ROWS_PER_TILE)],
                    acc_out.at[c, pl.ds(base, ROWS_PER_TILE)])


BLK = 512


def _dinv_body(parts_ref, dinv_ref):
    deg = jnp.sum(parts_ref[...], axis=0) + 1.0
    dinv_ref[...] = lax.rsqrt(deg)


def _dinv(parts):
    return pl.pallas_call(
        _dinv_body,
        out_shape=jax.ShapeDtypeStruct((NPAD // D, D), jnp.float32),
    )(parts)


def _xw_body(x_ref, w_ref, o_ref):
    xw = jnp.dot(x_ref[...], w_ref[...], preferred_element_type=jnp.float32)
    o_ref[0] = xw[:, :DH]
    o_ref[1] = xw[:, DH:]


def _xw_pass(x, w):
    return pl.pallas_call(
        _xw_body,
        grid=(NPAD // BLK,),
        in_specs=[
            pl.BlockSpec((BLK, D), lambda i: (i, 0)),
            pl.BlockSpec((D, D), lambda i: (0, 0)),
        ],
        out_specs=pl.BlockSpec((NC, BLK, DH), lambda i: (0, i, 0)),
        out_shape=jax.ShapeDtypeStruct((NC, NPAD, DH), jnp.float32),
    )(x, w)


def _scale_body(xw_ref, dinv_ref, y_ref):
    dinv = dinv_ref[...]
    y_ref[0] = dinv * xw_ref[0]
    y_ref[1] = dinv * xw_ref[1]


def _scale(xw, dinv_col):
    return pl.pallas_call(
        _scale_body,
        grid=(NPAD // BLK,),
        in_specs=[
            pl.BlockSpec((NC, BLK, DH), lambda i: (0, i, 0)),
            pl.BlockSpec((BLK, 1), lambda i: (i, 0)),
        ],
        out_specs=pl.BlockSpec((NC, BLK, DH), lambda i: (0, i, 0)),
        out_shape=jax.ShapeDtypeStruct((NC, NPAD, DH), jnp.float32),
    )(xw, dinv_col)


def _cm_body(acc_ref, y_ref, dinv_ref, b_ref, w_ref, o_ref):
    dinv = dinv_ref[...]
    b = b_ref[...]
    lo = dinv * (acc_ref[0] + y_ref[0]) + b[:, :DH]
    hi = dinv * (acc_ref[1] + y_ref[1]) + b[:, DH:]
    h = jnp.maximum(jnp.concatenate([lo, hi], axis=1), 0.0)
    y = dinv * jnp.dot(h, w_ref[...], preferred_element_type=jnp.float32)
    o_ref[0] = y[:, :DH]
    o_ref[1] = y[:, DH:]


def _combine_matmul_scale(acc, y, dinv_col, b, w):
    return pl.pallas_call(
        _cm_body,
        grid=(NPAD // BLK,),
        in_specs=[
            pl.BlockSpec((NC, BLK, DH), lambda i: (0, i, 0)),
            pl.BlockSpec((NC, BLK, DH), lambda i: (0, i, 0)),
            pl.BlockSpec((BLK, 1), lambda i: (i, 0)),
            pl.BlockSpec((1, D), lambda i: (0, 0)),
            pl.BlockSpec((D, D), lambda i: (0, 0)),
        ],
        out_specs=pl.BlockSpec((NC, BLK, DH), lambda i: (0, i, 0)),
        out_shape=jax.ShapeDtypeStruct((NC, NPAD, DH), jnp.float32),
    )(acc, y, dinv_col, b, w)


def _final_body(acc_ref, y_ref, dinv_ref, b_ref, o_ref):
    dinv = dinv_ref[...]
    b = b_ref[...]
    lo = dinv * (acc_ref[0] + y_ref[0]) + b[:, :DH]
    hi = dinv * (acc_ref[1] + y_ref[1]) + b[:, DH:]
    o_ref[...] = jnp.concatenate([lo, hi], axis=1)


def _final(acc, y, dinv_col, b):
    return pl.pallas_call(
        _final_body,
        grid=(NPAD // BLK,),
        in_specs=[
            pl.BlockSpec((NC, BLK, DH), lambda i: (0, i, 0)),
            pl.BlockSpec((NC, BLK, DH), lambda i: (0, i, 0)),
            pl.BlockSpec((BLK, 1), lambda i: (i, 0)),
            pl.BlockSpec((1, D), lambda i: (0, 0)),
        ],
        out_specs=pl.BlockSpec((BLK, D), lambda i: (i, 0)),
        out_shape=jax.ShapeDtypeStruct((NPAD, D), jnp.float32),
    )(acc, y, dinv_col, b)


def kernel(x, edge_index, W1, b1, W2, b2, W3, b3):
    src = edge_index[0].astype(jnp.int32)
    dst = edge_index[1].astype(jnp.int32)
    epad = EPAD - N_EDGES
    src_p = jnp.concatenate([src, jnp.zeros((epad,), jnp.int32)])
    dst_p = jnp.concatenate([dst, jnp.full((epad,), NPAD - 1, jnp.int32)])
    src16 = src_p.reshape(NS, NCHUNK, CHUNK)
    dst16 = dst_p.reshape(NS, NCHUNK, CHUNK)
    dst32 = dst_p.reshape(NC * NS, EPT32)

    x_pad = jnp.pad(x, ((0, NPAD - N_NODES), (0, 0)))

    xw1 = _xw_pass(x_pad, W1)
    deg_parts = _deg_kernel(dst32)
    dinv2d = _dinv(deg_parts.reshape(NC * NS, NPAD // D, D))
    dinv_col = dinv2d.reshape(NPAD, 1)

    y = _scale(xw1, dinv_col)
    acc = _segsum_kernel(y, src16, dst16)
    y = _combine_matmul_scale(acc, y, dinv_col, b1.reshape(1, D), W2)
    acc = _segsum_kernel(y, src16, dst16)
    y = _combine_matmul_scale(acc, y, dinv_col, b2.reshape(1, D), W3)
    acc = _segsum_kernel(y, src16, dst16)
    h = _final(acc, y, dinv_col, b3.reshape(1, D))
    return h[:N_NODES]


# speedup vs baseline: 20.1375x; 1.0817x over previous
import functools

import jax
import jax.numpy as jnp
from jax import lax
from jax.experimental import pallas as pl
from jax.experimental.pallas import tpu as pltpu
from jax.experimental.pallas import tpu_sc as plsc

N_NODES = 10000
D = 128
DH = D // 2
N_EDGES = 320000

NC = 2
NS = 16
CHUNK = 128
NPAD = 10240
EPAD = 327680
EPT32 = EPAD // (NC * NS)
EPT16 = EPAD // NS
NCHUNK = EPT16 // CHUNK
ROWS_PER_TILE = NPAD // NS
DEG_CHUNK = 2048

_MESH = plsc.VectorSubcoreMesh(core_axis_name="c", subcore_axis_name="s")
_SC_PARAMS = pltpu.CompilerParams(needs_layout_passes=False,
                                  use_tc_tiling_on_sc=False)


@functools.partial(
    pl.kernel,
    out_type=jax.ShapeDtypeStruct((NC * NS, NPAD), jnp.float32),
    mesh=_MESH,
    scratch_types=[
        pltpu.VMEM((DEG_CHUNK,), jnp.int32),
        pltpu.VMEM((NPAD,), jnp.float32),
    ],
    compiler_params=_SC_PARAMS,
)
def _deg_kernel(dst_hbm, deg_out, idx_v, deg_v):
    c = lax.axis_index("c")
    s = lax.axis_index("s")
    wid = c * NS + s
    zeros16 = jnp.zeros((16,), jnp.float32)
    ones16 = jnp.ones((16,), jnp.float32)

    def _zero(i, carry):
        deg_v[pl.ds(i * 16, 16)] = zeros16
        return carry

    lax.fori_loop(0, NPAD // 16, _zero, 0)

    def _count(i, carry):
        idx = idx_v[pl.ds(i * 16, 16)]
        plsc.addupdate_scatter(deg_v, [idx], ones16)
        return carry

    for ci in range(EPT32 // DEG_CHUNK):
        pltpu.sync_copy(dst_hbm.at[wid, pl.ds(ci * DEG_CHUNK, DEG_CHUNK)],
                        idx_v)
        lax.fori_loop(0, DEG_CHUNK // 16, _count, 0)
    pltpu.sync_copy(deg_v, deg_out.at[wid])


PHCH = 10
NPHASE = NCHUNK // PHCH


@functools.partial(
    pl.kernel,
    out_type=jax.ShapeDtypeStruct((NC, NPAD, DH), jnp.float32),
    mesh=_MESH,
    scratch_types=[
        pltpu.VMEM((2, PHCH, CHUNK), jnp.int32),
        pltpu.VMEM((2, PHCH, CHUNK), jnp.int32),
        pltpu.VMEM((4, CHUNK, DH), jnp.float32),
        pltpu.VMEM_SHARED((NPAD, DH), jnp.float32),
        pltpu.VMEM_SHARED((NPAD, DH), jnp.float32),
        pltpu.SemaphoreType.DMA,
        pltpu.SemaphoreType.DMA,
        pltpu.SemaphoreType.DMA,
        pltpu.SemaphoreType.DMA,
        pltpu.SemaphoreType.DMA,
        pltpu.SemaphoreType.DMA,
        pltpu.SemaphoreType.DMA,
        pltpu.SemaphoreType.DMA,
        pltpu.SemaphoreType.DMA,
        pltpu.SemaphoreType.DMA,
    ],
    compiler_params=_SC_PARAMS,
)
def _segsum_kernel(y_hbm, src_hbm, dst_hbm, acc_out,
                   src_v, dst_v, rows_v, y_sh, acc_sh,
                   g0, g1, g2, g3, s0, s1, s2, s3, i0, i1):
    c = lax.axis_index("c")
    s = lax.axis_index("s")
    gsem = (g0, g1, g2, g3)
    ssem = (s0, s1, s2, s3)
    isem = (i0, i1)

    def _idx_fetch(p, pb):
        pltpu.async_copy(src_hbm.at[s, pl.ds(p * PHCH, PHCH)],
                         src_v.at[pb], isem[pb])
        pltpu.async_copy(dst_hbm.at[s, pl.ds(p * PHCH, PHCH)],
                         dst_v.at[pb], isem[pb])

    def _idx_wait(pb):
        pltpu.make_async_copy(src_hbm.at[s, pl.ds(0, PHCH)],
                              src_v.at[pb], isem[pb]).wait()
        pltpu.make_async_copy(dst_hbm.at[s, pl.ds(0, PHCH)],
                              dst_v.at[pb], isem[pb]).wait()

    _idx_fetch(0, 0)
    _idx_fetch(1, 1)

    zeros16 = jnp.zeros((16,), jnp.float32)

    def _zero(i, carry):
        r = i // (DH // 16)
        k = i % (DH // 16)
        rows_v[0, r, pl.ds(k * 16, 16)] = zeros16
        return carry

    lax.fori_loop(0, CHUNK * DH // 16, _zero, 0)

    base = s * ROWS_PER_TILE
    for z in range(ROWS_PER_TILE // CHUNK):
        pltpu.sync_copy(rows_v.at[0], acc_sh.at[pl.ds(base + z * CHUNK, CHUNK)])
    pltpu.sync_copy(y_hbm.at[c, pl.ds(base, ROWS_PER_TILE)],
                    y_sh.at[pl.ds(base, ROWS_PER_TILE)])
    plsc.subcore_barrier()

    def _gather(idx_row, b):
        pltpu.async_copy(y_sh.at[idx_row], rows_v.at[b], gsem[b])

    def _wait_gather(b):
        pltpu.make_async_copy(y_sh.at[src_v.at[0, 0]], rows_v.at[b],
                              gsem[b]).wait()

    def _scatter(dst_row, b):
        pltpu.async_copy(rows_v.at[b], acc_sh.at[dst_row], ssem[b], add=True)

    def _wait_scatter(b):
        pltpu.make_async_copy(rows_v.at[b], acc_sh.at[dst_v.at[0, 0]],
                              ssem[b]).wait()

    def _steady(pb, jj):
        b = (jj + 2 * pb) % 4
        _wait_gather(b)
        if jj >= PHCH - 2:
            pltpu.sync_copy(rows_v.at[b], acc_sh.at[dst_v.at[pb, jj]],
                            add=True)
        else:
            _scatter(dst_v.at[pb, jj], b)
        if jj >= 2:
            _wait_scatter((b + 2) % 4)
        if jj + 2 < PHCH:
            _gather(src_v.at[pb, jj + 2], (b + 2) % 4)
        else:
            _gather(src_v.at[1 - pb, jj + 2 - PHCH], (b + 2) % 4)

    def _phase_chunks(pb, jj_lo=0, jj_hi=PHCH):
        for jj in range(jj_lo, jj_hi):
            _steady(pb, jj)

    _idx_wait(0)
    _gather(src_v.at[0, 0], 0)
    _gather(src_v.at[0, 1], 1)
    _idx_wait(1)
    for k in (0, 1):
        _wait_gather(k)
        _scatter(dst_v.at[0, k], k)
        _gather(src_v.at[0, k + 2], k + 2)
    _phase_chunks(0, jj_lo=2)
    _idx_fetch(2, 0)

    def _superphase(q, carry):
        p = 1 + 2 * q
        _idx_wait(0)
        _phase_chunks(1)
        _idx_fetch(p + 2, 1)
        _idx_wait(1)
        _phase_chunks(0)
        _idx_fetch(p + 3, 0)
        return carry

    lax.fori_loop(0, 6, _superphase, 0)
    _idx_wait(0)
    _phase_chunks(1)
    _idx_fetch(15, 1)
    _idx_wait(1)
    _phase_chunks(0)
    _phase_chunks(1, jj_hi=PHCH - 2)
    for k in (PHCH - 2, PHCH - 1):
        b = (k + 2) % 4
        _wait_gather(b)
        pltpu.sync_copy(rows_v.at[b], acc_sh.at[dst_v.at[1, k]], add=True)
        _wait_scatter((b + 2) % 4)
    plsc.subcore_barrier()

    pltpu.sync_copy(acc_sh.at[pl.ds(base, ROWS_PER_TILE)],
                    acc_out.at[c, pl.ds(base, ROWS_PER_TILE)])


BLK = 512


def _dinv_body(parts_ref, dinv_ref):
    deg = jnp.sum(parts_ref[...], axis=0) + 1.0
    dinv_ref[...] = lax.rsqrt(deg)


def _dinv(parts):
    return pl.pallas_call(
        _dinv_body,
        out_shape=jax.ShapeDtypeStruct((NPAD // D, D), jnp.float32),
    )(parts)


def _xw_body(x_ref, w_ref, o_ref):
    xw = jnp.dot(x_ref[...], w_ref[...], preferred_element_type=jnp.float32)
    o_ref[0] = xw[:, :DH]
    o_ref[1] = xw[:, DH:]


def _xw_pass(x, w):
    return pl.pallas_call(
        _xw_body,
        grid=(NPAD // BLK,),
        in_specs=[
            pl.BlockSpec((BLK, D), lambda i: (i, 0)),
            pl.BlockSpec((D, D), lambda i: (0, 0)),
        ],
        out_specs=pl.BlockSpec((NC, BLK, DH), lambda i: (0, i, 0)),
        out_shape=jax.ShapeDtypeStruct((NC, NPAD, DH), jnp.float32),
    )(x, w)


def _scale_body(xw_ref, dinv_ref, y_ref):
    dinv = dinv_ref[...]
    y_ref[0] = dinv * xw_ref[0]
    y_ref[1] = dinv * xw_ref[1]


def _scale(xw, dinv_col):
    return pl.pallas_call(
        _scale_body,
        grid=(NPAD // BLK,),
        in_specs=[
            pl.BlockSpec((NC, BLK, DH), lambda i: (0, i, 0)),
            pl.BlockSpec((BLK, 1), lambda i: (i, 0)),
        ],
        out_specs=pl.BlockSpec((NC, BLK, DH), lambda i: (0, i, 0)),
        out_shape=jax.ShapeDtypeStruct((NC, NPAD, DH), jnp.float32),
    )(xw, dinv_col)


def _cm_body(acc_ref, y_ref, dinv_ref, b_ref, w_ref, o_ref):
    dinv = dinv_ref[...]
    b = b_ref[...]
    lo = dinv * (acc_ref[0] + y_ref[0]) + b[:, :DH]
    hi = dinv * (acc_ref[1] + y_ref[1]) + b[:, DH:]
    h = jnp.maximum(jnp.concatenate([lo, hi], axis=1), 0.0)
    y = dinv * jnp.dot(h, w_ref[...], preferred_element_type=jnp.float32)
    o_ref[0] = y[:, :DH]
    o_ref[1] = y[:, DH:]


def _combine_matmul_scale(acc, y, dinv_col, b, w):
    return pl.pallas_call(
        _cm_body,
        grid=(NPAD // BLK,),
        in_specs=[
            pl.BlockSpec((NC, BLK, DH), lambda i: (0, i, 0)),
            pl.BlockSpec((NC, BLK, DH), lambda i: (0, i, 0)),
            pl.BlockSpec((BLK, 1), lambda i: (i, 0)),
            pl.BlockSpec((1, D), lambda i: (0, 0)),
            pl.BlockSpec((D, D), lambda i: (0, 0)),
        ],
        out_specs=pl.BlockSpec((NC, BLK, DH), lambda i: (0, i, 0)),
        out_shape=jax.ShapeDtypeStruct((NC, NPAD, DH), jnp.float32),
    )(acc, y, dinv_col, b, w)


def _final_body(acc_ref, y_ref, dinv_ref, b_ref, o_ref):
    dinv = dinv_ref[...]
    b = b_ref[...]
    lo = dinv * (acc_ref[0] + y_ref[0]) + b[:, :DH]
    hi = dinv * (acc_ref[1] + y_ref[1]) + b[:, DH:]
    o_ref[...] = jnp.concatenate([lo, hi], axis=1)


def _final(acc, y, dinv_col, b):
    return pl.pallas_call(
        _final_body,
        grid=(NPAD // BLK,),
        in_specs=[
            pl.BlockSpec((NC, BLK, DH), lambda i: (0, i, 0)),
            pl.BlockSpec((NC, BLK, DH), lambda i: (0, i, 0)),
            pl.BlockSpec((BLK, 1), lambda i: (i, 0)),
            pl.BlockSpec((1, D), lambda i: (0, 0)),
        ],
        out_specs=pl.BlockSpec((BLK, D), lambda i: (i, 0)),
        out_shape=jax.ShapeDtypeStruct((NPAD, D), jnp.float32),
    )(acc, y, dinv_col, b)


def kernel(x, edge_index, W1, b1, W2, b2, W3, b3):
    src = edge_index[0].astype(jnp.int32)
    dst = edge_index[1].astype(jnp.int32)
    epad = EPAD - N_EDGES
    src_p = jnp.concatenate([src, jnp.zeros((epad,), jnp.int32)])
    dst_p = jnp.concatenate([dst, jnp.full((epad,), NPAD - 1, jnp.int32)])
    src16 = src_p.reshape(NS, NCHUNK, CHUNK)
    dst16 = dst_p.reshape(NS, NCHUNK, CHUNK)
    dst32 = dst_p.reshape(NC * NS, EPT32)

    x_pad = jnp.pad(x, ((0, NPAD - N_NODES), (0, 0)))

    xw1 = _xw_pass(x_pad, W1)
    deg_parts = _deg_kernel(dst32)
    dinv2d = _dinv(deg_parts.reshape(NC * NS, NPAD // D, D))
    dinv_col = dinv2d.reshape(NPAD, 1)

    y = _scale(xw1, dinv_col)
    acc = _segsum_kernel(y, src16, dst16)
    y = _combine_matmul_scale(acc, y, dinv_col, b1.reshape(1, D), W2)
    acc = _segsum_kernel(y, src16, dst16)
    y = _combine_matmul_scale(acc, y, dinv_col, b2.reshape(1, D), W3)
    acc = _segsum_kernel(y, src16, dst16)
    h = _final(acc, y, dinv_col, b3.reshape(1, D))
    return h[:N_NODES]


# speedup vs baseline: 20.8834x; 1.0370x over previous
import functools

import jax
import jax.numpy as jnp
from jax import lax
from jax.experimental import pallas as pl
from jax.experimental.pallas import tpu as pltpu
from jax.experimental.pallas import tpu_sc as plsc

N_NODES = 10000
D = 128
DH = D // 2
N_EDGES = 320000

NC = 2
NS = 16
CHUNK = 128
NPAD = 10240
EPAD = 327680
EPT32 = EPAD // (NC * NS)
EPT16 = EPAD // NS
NCHUNK = EPT16 // CHUNK
ROWS_PER_TILE = NPAD // NS
DEG_CHUNK = 2048

_MESH = plsc.VectorSubcoreMesh(core_axis_name="c", subcore_axis_name="s")
_SC_PARAMS = pltpu.CompilerParams(needs_layout_passes=False,
                                  use_tc_tiling_on_sc=False)


@functools.partial(
    pl.kernel,
    out_type=jax.ShapeDtypeStruct((NC * NS, NPAD), jnp.float32),
    mesh=_MESH,
    scratch_types=[
        pltpu.VMEM((DEG_CHUNK,), jnp.int32),
        pltpu.VMEM((NPAD,), jnp.float32),
    ],
    compiler_params=_SC_PARAMS,
)
def _deg_kernel(dst_hbm, deg_out, idx_v, deg_v):
    c = lax.axis_index("c")
    s = lax.axis_index("s")
    wid = c * NS + s
    zeros16 = jnp.zeros((16,), jnp.float32)
    ones16 = jnp.ones((16,), jnp.float32)

    def _zero(i, carry):
        deg_v[pl.ds(i * 16, 16)] = zeros16
        return carry

    lax.fori_loop(0, NPAD // 16, _zero, 0)

    def _count(i, carry):
        idx = idx_v[pl.ds(i * 16, 16)]
        plsc.addupdate_scatter(deg_v, [idx], ones16)
        return carry

    for ci in range(EPT32 // DEG_CHUNK):
        pltpu.sync_copy(dst_hbm.at[wid, pl.ds(ci * DEG_CHUNK, DEG_CHUNK)],
                        idx_v)
        lax.fori_loop(0, DEG_CHUNK // 16, _count, 0)
    pltpu.sync_copy(deg_v, deg_out.at[wid])


PHCH = 10
NPHASE = NCHUNK // PHCH


@functools.partial(
    pl.kernel,
    out_type=jax.ShapeDtypeStruct((NC, NPAD, DH), jnp.float32),
    mesh=_MESH,
    scratch_types=[
        pltpu.VMEM((2, PHCH, CHUNK), jnp.int32),
        pltpu.VMEM((2, PHCH, CHUNK), jnp.int32),
        pltpu.VMEM((4, CHUNK, DH), jnp.float32),
        pltpu.VMEM_SHARED((NPAD, DH), jnp.float32),
        pltpu.VMEM_SHARED((NPAD, DH), jnp.float32),
        pltpu.SemaphoreType.DMA,
        pltpu.SemaphoreType.DMA,
        pltpu.SemaphoreType.DMA,
        pltpu.SemaphoreType.DMA,
        pltpu.SemaphoreType.DMA,
        pltpu.SemaphoreType.DMA,
        pltpu.SemaphoreType.DMA,
        pltpu.SemaphoreType.DMA,
        pltpu.SemaphoreType.DMA,
        pltpu.SemaphoreType.DMA,
    ],
    compiler_params=_SC_PARAMS,
)
def _segsum_kernel(y_hbm, src_hbm, dst_hbm, acc_out,
                   src_v, dst_v, rows_v, y_sh, acc_sh,
                   g0, g1, g2, g3, s0, s1, s2, s3, i0, i1):
    c = lax.axis_index("c")
    s = lax.axis_index("s")
    gsem = (g0, g1, g2, g3)
    ssem = (s0, s1, s2, s3)
    isem = (i0, i1)

    def _idx_fetch(p, pb):
        pltpu.async_copy(src_hbm.at[s, pl.ds(p * PHCH, PHCH)],
                         src_v.at[pb], isem[pb])
        pltpu.async_copy(dst_hbm.at[s, pl.ds(p * PHCH, PHCH)],
                         dst_v.at[pb], isem[pb])

    def _idx_wait(pb):
        pltpu.make_async_copy(src_hbm.at[s, pl.ds(0, PHCH)],
                              src_v.at[pb], isem[pb]).wait()
        pltpu.make_async_copy(dst_hbm.at[s, pl.ds(0, PHCH)],
                              dst_v.at[pb], isem[pb]).wait()

    _idx_fetch(0, 0)
    _idx_fetch(1, 1)

    zeros16 = jnp.zeros((16,), jnp.float32)

    def _zero(i, carry):
        r = i // (DH // 16)
        k = i % (DH // 16)
        rows_v[0, r, pl.ds(k * 16, 16)] = zeros16
        return carry

    lax.fori_loop(0, CHUNK * DH // 16, _zero, 0)

    base = s * ROWS_PER_TILE
    for z in range(ROWS_PER_TILE // CHUNK):
        pltpu.sync_copy(rows_v.at[0], acc_sh.at[pl.ds(base + z * CHUNK, CHUNK)])
    pltpu.sync_copy(y_hbm.at[c, pl.ds(base, ROWS_PER_TILE)],
                    y_sh.at[pl.ds(base, ROWS_PER_TILE)])
    plsc.subcore_barrier()

    def _gather(idx_row, b):
        pltpu.async_copy(y_sh.at[idx_row], rows_v.at[b], gsem[b])

    def _wait_gather(b):
        pltpu.make_async_copy(y_sh.at[src_v.at[0, 0]], rows_v.at[b],
                              gsem[b]).wait()

    def _scatter(dst_row, b):
        pltpu.async_copy(rows_v.at[b], acc_sh.at[dst_row], ssem[b], add=True)

    def _wait_scatter(b):
        pltpu.make_async_copy(rows_v.at[b], acc_sh.at[dst_v.at[0, 0]],
                              ssem[b]).wait()

    def _steady(pb, jj):
        b = (jj + 2 * pb) % 4
        _wait_gather(b)
        _scatter(dst_v.at[pb, jj], b)
        if jj >= 2:
            _wait_scatter((b + 2) % 4)
        if jj + 2 < PHCH:
            _gather(src_v.at[pb, jj + 2], (b + 2) % 4)
        else:
            _gather(src_v.at[1 - pb, jj + 2 - PHCH], (b + 2) % 4)

    def _phase_chunks(pb, jj_lo=0, jj_hi=PHCH):
        for jj in range(jj_lo, jj_hi):
            _steady(pb, jj)
        if jj_hi == PHCH:
            _wait_scatter((PHCH - 2 + 2 * pb) % 4)
            _wait_scatter((PHCH - 1 + 2 * pb) % 4)

    _idx_wait(0)
    _gather(src_v.at[0, 0], 0)
    _gather(src_v.at[0, 1], 1)
    _idx_wait(1)
    for k in (0, 1):
        _wait_gather(k)
        _scatter(dst_v.at[0, k], k)
        _gather(src_v.at[0, k + 2], k + 2)
    _phase_chunks(0, jj_lo=2)
    _idx_fetch(2, 0)

    def _superphase(q, carry):
        p = 1 + 2 * q
        _idx_wait(0)
        _phase_chunks(1)
        _idx_fetch(p + 2, 1)
        _idx_wait(1)
        _phase_chunks(0)
        _idx_fetch(p + 3, 0)
        return carry

    lax.fori_loop(0, 6, _superphase, 0)
    _idx_wait(0)
    _phase_chunks(1)
    _idx_fetch(15, 1)
    _idx_wait(1)
    _phase_chunks(0)
    _phase_chunks(1, jj_hi=PHCH - 2)
    for k in (PHCH - 2, PHCH - 1):
        b = (k + 2) % 4
        _wait_gather(b)
        pltpu.sync_copy(rows_v.at[b], acc_sh.at[dst_v.at[1, k]], add=True)
        _wait_scatter((b + 2) % 4)
    plsc.subcore_barrier()

    pltpu.sync_copy(acc_sh.at[pl.ds(base, ROWS_PER_TILE)],
                    acc_out.at[c, pl.ds(base, ROWS_PER_TILE)])


BLK = 512


def _dinv_body(parts_ref, dinv_ref):
    deg = jnp.sum(parts_ref[...], axis=0) + 1.0
    dinv_ref[...] = lax.rsqrt(deg)


def _dinv(parts):
    return pl.pallas_call(
        _dinv_body,
        out_shape=jax.ShapeDtypeStruct((NPAD // D, D), jnp.float32),
    )(parts)


def _xw_body(x_ref, w_ref, o_ref):
    xw = jnp.dot(x_ref[...], w_ref[...], preferred_element_type=jnp.float32)
    o_ref[0] = xw[:, :DH]
    o_ref[1] = xw[:, DH:]


def _xw_pass(x, w):
    return pl.pallas_call(
        _xw_body,
        grid=(NPAD // BLK,),
        in_specs=[
            pl.BlockSpec((BLK, D), lambda i: (i, 0)),
            pl.BlockSpec((D, D), lambda i: (0, 0)),
        ],
        out_specs=pl.BlockSpec((NC, BLK, DH), lambda i: (0, i, 0)),
        out_shape=jax.ShapeDtypeStruct((NC, NPAD, DH), jnp.float32),
    )(x, w)


def _scale_body(xw_ref, dinv_ref, y_ref):
    dinv = dinv_ref[...]
    y_ref[0] = dinv * xw_ref[0]
    y_ref[1] = dinv * xw_ref[1]


def _scale(xw, dinv_col):
    return pl.pallas_call(
        _scale_body,
        grid=(NPAD // BLK,),
        in_specs=[
            pl.BlockSpec((NC, BLK, DH), lambda i: (0, i, 0)),
            pl.BlockSpec((BLK, 1), lambda i: (i, 0)),
        ],
        out_specs=pl.BlockSpec((NC, BLK, DH), lambda i: (0, i, 0)),
        out_shape=jax.ShapeDtypeStruct((NC, NPAD, DH), jnp.float32),
    )(xw, dinv_col)


def _cm_body(acc_ref, y_ref, dinv_ref, b_ref, w_ref, o_ref):
    dinv = dinv_ref[...]
    b = b_ref[...]
    lo = dinv * (acc_ref[0] + y_ref[0]) + b[:, :DH]
    hi = dinv * (acc_ref[1] + y_ref[1]) + b[:, DH:]
    h = jnp.maximum(jnp.concatenate([lo, hi], axis=1), 0.0)
    y = dinv * jnp.dot(h, w_ref[...], preferred_element_type=jnp.float32)
    o_ref[0] = y[:, :DH]
    o_ref[1] = y[:, DH:]


def _combine_matmul_scale(acc, y, dinv_col, b, w):
    return pl.pallas_call(
        _cm_body,
        grid=(NPAD // BLK,),
        in_specs=[
            pl.BlockSpec((NC, BLK, DH), lambda i: (0, i, 0)),
            pl.BlockSpec((NC, BLK, DH), lambda i: (0, i, 0)),
            pl.BlockSpec((BLK, 1), lambda i: (i, 0)),
            pl.BlockSpec((1, D), lambda i: (0, 0)),
            pl.BlockSpec((D, D), lambda i: (0, 0)),
        ],
        out_specs=pl.BlockSpec((NC, BLK, DH), lambda i: (0, i, 0)),
        out_shape=jax.ShapeDtypeStruct((NC, NPAD, DH), jnp.float32),
    )(acc, y, dinv_col, b, w)


def _final_body(acc_ref, y_ref, dinv_ref, b_ref, o_ref):
    dinv = dinv_ref[...]
    b = b_ref[...]
    lo = dinv * (acc_ref[0] + y_ref[0]) + b[:, :DH]
    hi = dinv * (acc_ref[1] + y_ref[1]) + b[:, DH:]
    o_ref[...] = jnp.concatenate([lo, hi], axis=1)


def _final(acc, y, dinv_col, b):
    return pl.pallas_call(
        _final_body,
        grid=(NPAD // BLK,),
        in_specs=[
            pl.BlockSpec((NC, BLK, DH), lambda i: (0, i, 0)),
            pl.BlockSpec((NC, BLK, DH), lambda i: (0, i, 0)),
            pl.BlockSpec((BLK, 1), lambda i: (i, 0)),
            pl.BlockSpec((1, D), lambda i: (0, 0)),
        ],
        out_specs=pl.BlockSpec((BLK, D), lambda i: (i, 0)),
        out_shape=jax.ShapeDtypeStruct((NPAD, D), jnp.float32),
    )(acc, y, dinv_col, b)


def kernel(x, edge_index, W1, b1, W2, b2, W3, b3):
    src = edge_index[0].astype(jnp.int32)
    dst = edge_index[1].astype(jnp.int32)
    epad = EPAD - N_EDGES
    src_p = jnp.concatenate([src, jnp.zeros((epad,), jnp.int32)])
    dst_p = jnp.concatenate([dst, jnp.full((epad,), NPAD - 1, jnp.int32)])
    src16 = src_p.reshape(NS, NCHUNK, CHUNK)
    dst16 = dst_p.reshape(NS, NCHUNK, CHUNK)
    dst32 = dst_p.reshape(NC * NS, EPT32)

    x_pad = jnp.pad(x, ((0, NPAD - N_NODES), (0, 0)))

    xw1 = _xw_pass(x_pad, W1)
    deg_parts = _deg_kernel(dst32)
    dinv2d = _dinv(deg_parts.reshape(NC * NS, NPAD // D, D))
    dinv_col = dinv2d.reshape(NPAD, 1)

    y = _scale(xw1, dinv_col)
    acc = _segsum_kernel(y, src16, dst16)
    y = _combine_matmul_scale(acc, y, dinv_col, b1.reshape(1, D), W2)
    acc = _segsum_kernel(y, src16, dst16)
    y = _combine_matmul_scale(acc, y, dinv_col, b2.reshape(1, D), W3)
    acc = _segsum_kernel(y, src16, dst16)
    h = _final(acc, y, dinv_col, b3.reshape(1, D))
    return h[:N_NODES]
